# baseline (speedup 1.0000x reference)
import functools

import jax
import jax.numpy as jnp
from jax import lax
from jax.experimental import pallas as pl
from jax.experimental.pallas import tpu as pltpu, tpu_sc as plsc

NUM_ROWS = 1024
ROW_LEN = 8193
PAD_LEN = 8208
NUM_CHUNKS = PAD_LEN // 16
NC, NS = 2, 16
NW = NC * NS
ROWS_PER_W = NUM_ROWS // NW
NUM_BINS_K = 16


def _sc_body(h_hbm, s_hbm, out_hbm, hbuf, sbuf, binsbuf, outbuf):
    wid = lax.axis_index("s") * NC + lax.axis_index("c")
    base_row = wid * ROWS_PER_W
    zeros16 = jnp.zeros((16,), jnp.float32)
    iota16 = lax.broadcasted_iota(jnp.int32, (16,), 0)

    hbuf[pl.ds(ROW_LEN - 1, 16)] = zeros16
    sbuf[pl.ds(ROW_LEN - 1, 16)] = zeros16

    def row_body(i, carry):
        acc0, acc1 = carry
        row = base_row + i
        pltpu.sync_copy(h_hbm.at[row], hbuf.at[pl.ds(0, ROW_LEN)])
        pltpu.sync_copy(s_hbm.at[row], sbuf.at[pl.ds(0, ROW_LEN)])

        def pass_a(c, acc):
            hv = hbuf[pl.ds(c * 16, 16)]
            return acc + hv * hv

        acc = lax.fori_loop(0, NUM_CHUNKS, pass_a, zeros16)
        total_a = jnp.sum(acc)
        inv_vec = jnp.full((16,), 16.0) / jnp.full((16,), total_a)

        binsbuf[...] = zeros16

        def pass_b(c, carry_b):
            cb, ut = carry_b
            hv = hbuf[pl.ds(c * 16, 16)]
            sv = sbuf[pl.ds(c * 16, 16)]
            xv = hv * hv
            pc = plsc.cumsum(xv)
            p_incl = cb + pc
            p_excl = p_incl - xv
            scaled = p_excl * inv_vec
            k = jnp.minimum(scaled.astype(jnp.int32), NUM_BINS_K - 1)
            m = scaled > 0.0
            inc = hv * sv
            plsc.addupdate_scatter(binsbuf, [k], inc, mask=m)
            return cb + jnp.sum(xv), ut + inc

        _, ut = lax.fori_loop(0, NUM_CHUNKS, pass_b, (zeros16, zeros16))

        u_tot = jnp.sum(ut)
        sv_bins = binsbuf[...]
        dev = sv_bins - u_tot * (1.0 / 16.0)
        css = jnp.sum(dev * dev)
        chisq_b = jnp.full((16,), css) * inv_vec * ((16.0 / 15.0) / 32.0)
        m0 = (iota16 == i) & (i < 16)
        m1 = (iota16 == (i - 16)) & (i >= 16)
        acc0 = acc0 + jnp.where(m0, chisq_b, zeros16)
        acc1 = acc1 + jnp.where(m1, chisq_b, zeros16)
        return acc0, acc1

    acc0, acc1 = lax.fori_loop(0, ROWS_PER_W, row_body, (zeros16, zeros16))
    outbuf[pl.ds(0, 16)] = acc0
    outbuf[pl.ds(16, 16)] = acc1
    pltpu.sync_copy(outbuf, out_hbm.at[pl.ds(base_row, ROWS_PER_W)])


@jax.jit
def kernel(template, strain):
    h = template.reshape(NUM_ROWS, ROW_LEN)
    s = strain.reshape(NUM_ROWS, ROW_LEN)
    mesh = plsc.VectorSubcoreMesh(
        core_axis_name="c", subcore_axis_name="s", num_cores=NC, num_subcores=NS
    )
    out = pl.kernel(
        _sc_body,
        out_type=jax.ShapeDtypeStruct((NUM_ROWS,), jnp.float32),
        mesh=mesh,
        scratch_types=[
            pltpu.VMEM((PAD_LEN,), jnp.float32),
            pltpu.VMEM((PAD_LEN,), jnp.float32),
            pltpu.VMEM((16,), jnp.float32),
            pltpu.VMEM((ROWS_PER_W,), jnp.float32),
        ],
        compiler_params=pltpu.CompilerParams(
            needs_layout_passes=False, use_tc_tiling_on_sc=False
        ),
    )(h, s)
    return out.reshape(512, 2)


# speedup vs baseline: 2.1362x; 2.1362x over previous
import jax
import jax.numpy as jnp
from jax import lax
from jax.experimental import pallas as pl
from jax.experimental.pallas import tpu as pltpu, tpu_sc as plsc

NUM_ROWS = 1024
ROW_LEN = 8193
SEG_LEN = 515
PAD_LEN = 16 * SEG_LEN
UNROLL = 5
NUM_STEPS = SEG_LEN // UNROLL
NC, NS = 2, 16
NW = NC * NS
ROWS_PER_W = NUM_ROWS // NW


def _sc_body(h_hbm, s_hbm, out_hbm, h0, s0, h1, s1, binsbuf, outbuf, sem0, sem1):
    wid = lax.axis_index("s") * NC + lax.axis_index("c")
    base_row = wid * ROWS_PER_W
    zeros16 = jnp.zeros((16,), jnp.float32)
    iota16 = lax.broadcasted_iota(jnp.int32, (16,), 0)
    vidx0 = iota16 * SEG_LEN

    for buf in (h0, s0, h1, s1):
        for off in range(ROW_LEN - 1, PAD_LEN - 15, 16):
            buf[pl.ds(off, 16)] = zeros16

    def start_row(row, hbuf, sbuf, sem):
        pltpu.make_async_copy(h_hbm.at[row], hbuf.at[pl.ds(0, ROW_LEN)], sem).start()
        pltpu.make_async_copy(s_hbm.at[row], sbuf.at[pl.ds(0, ROW_LEN)], sem).start()

    def wait_row(row, hbuf, sbuf, sem):
        pltpu.make_async_copy(h_hbm.at[row], hbuf.at[pl.ds(0, ROW_LEN)], sem).wait()
        pltpu.make_async_copy(s_hbm.at[row], sbuf.at[pl.ds(0, ROW_LEN)], sem).wait()

    def process_row(i2, hbuf, sbuf, acc0, acc1):
        def pass_a(_, carry):
            acc, vidx = carry
            for _u in range(UNROLL):
                hv = plsc.load_gather(hbuf, [vidx])
                acc = acc + hv * hv
                vidx = vidx + 1
            return acc, vidx

        seg_sums, _ = lax.fori_loop(0, NUM_STEPS, pass_a, (zeros16, vidx0))
        seg_incl = plsc.cumsum(seg_sums)
        seg_base = seg_incl - seg_sums
        a_vec = jnp.full((16,), jnp.sum(seg_sums))
        inv_vec = jnp.full((16,), 16.0) / a_vec

        binsbuf[pl.ds(0, 16)] = zeros16

        def pass_b(_, carry):
            p_run, ut, vidx = carry
            for _u in range(UNROLL):
                hv = plsc.load_gather(hbuf, [vidx])
                sv = plsc.load_gather(sbuf, [vidx])
                xv = hv * hv
                scaled = p_run * inv_vec
                k = scaled.astype(jnp.int32)
                m = scaled > 0.0
                inc = hv * sv
                plsc.addupdate_scatter(binsbuf, [k], inc, mask=m)
                p_run = p_run + xv
                ut = ut + inc
                vidx = vidx + 1
            return p_run, ut, vidx

        _, ut, _ = lax.fori_loop(0, NUM_STEPS, pass_b, (seg_base, zeros16, vidx0))

        u_tot = jnp.full((16,), jnp.sum(ut))
        sv_bins = binsbuf[pl.ds(0, 16)]
        dev = sv_bins - u_tot * (1.0 / 16.0)
        css = jnp.sum(dev * dev)
        chisq_b = jnp.full((16,), css) * inv_vec * ((16.0 / 15.0) / 32.0)
        acc0 = acc0 + jnp.where((iota16 == i2) & (i2 < 16), chisq_b, zeros16)
        acc1 = acc1 + jnp.where((iota16 == (i2 - 16)) & (i2 >= 16), chisq_b, zeros16)
        return acc0, acc1

    start_row(base_row, h0, s0, sem0)

    def pair_body(i, carry):
        acc0, acc1 = carry
        ra = base_row + 2 * i
        rb = ra + 1
        start_row(rb, h1, s1, sem1)
        wait_row(ra, h0, s0, sem0)
        acc0, acc1 = process_row(2 * i, h0, s0, acc0, acc1)

        @pl.when(i < (ROWS_PER_W // 2 - 1))
        def _():
            start_row(ra + 2, h0, s0, sem0)

        wait_row(rb, h1, s1, sem1)
        acc0, acc1 = process_row(2 * i + 1, h1, s1, acc0, acc1)
        return acc0, acc1

    acc0, acc1 = lax.fori_loop(0, ROWS_PER_W // 2, pair_body, (zeros16, zeros16))
    outbuf[pl.ds(0, 16)] = acc0
    outbuf[pl.ds(16, 16)] = acc1
    pltpu.sync_copy(outbuf, out_hbm.at[pl.ds(base_row, ROWS_PER_W)])


@jax.jit
def kernel(template, strain):
    h = template.reshape(NUM_ROWS, ROW_LEN)
    s = strain.reshape(NUM_ROWS, ROW_LEN)
    mesh = plsc.VectorSubcoreMesh(
        core_axis_name="c", subcore_axis_name="s", num_cores=NC, num_subcores=NS
    )
    out = pl.kernel(
        _sc_body,
        out_type=jax.ShapeDtypeStruct((NUM_ROWS,), jnp.float32),
        mesh=mesh,
        scratch_types=[
            pltpu.VMEM((PAD_LEN,), jnp.float32),
            pltpu.VMEM((PAD_LEN,), jnp.float32),
            pltpu.VMEM((PAD_LEN,), jnp.float32),
            pltpu.VMEM((PAD_LEN,), jnp.float32),
            pltpu.VMEM((32,), jnp.float32),
            pltpu.VMEM((ROWS_PER_W,), jnp.float32),
            pltpu.SemaphoreType.DMA,
            pltpu.SemaphoreType.DMA,
        ],
        compiler_params=pltpu.CompilerParams(
            needs_layout_passes=False, use_tc_tiling_on_sc=False
        ),
    )(h, s)
    return out.reshape(512, 2)


# speedup vs baseline: 2.3389x; 1.0949x over previous
import jax
import jax.numpy as jnp
from jax import lax
from jax.experimental import pallas as pl
from jax.experimental.pallas import tpu as pltpu, tpu_sc as plsc

NUM_ROWS = 1024
ROW_LEN = 8193
SEG_LEN = 515
PAD_LEN = 16 * SEG_LEN
UNROLL = 5
NUM_STEPS = SEG_LEN // UNROLL
NC, NS = 2, 16
NW = NC * NS
ROWS_PER_W = NUM_ROWS // NW


def _sc_body(h_hbm, s_hbm, out_hbm, h0, s0, h1, s1, binsbuf, outbuf, sem0, sem1):
    wid = lax.axis_index("s") * NC + lax.axis_index("c")
    base_row = wid * ROWS_PER_W
    zeros16 = jnp.zeros((16,), jnp.float32)
    iota16 = lax.broadcasted_iota(jnp.int32, (16,), 0)
    vidx0 = iota16 * SEG_LEN

    for buf in (h0, s0, h1, s1):
        for off in range(ROW_LEN - 1, PAD_LEN - 15, 16):
            buf[pl.ds(off, 16)] = zeros16

    def start_row(row, hbuf, sbuf, sem):
        pltpu.make_async_copy(h_hbm.at[row], hbuf.at[pl.ds(0, ROW_LEN)], sem).start()
        pltpu.make_async_copy(s_hbm.at[row], sbuf.at[pl.ds(0, ROW_LEN)], sem).start()

    def wait_row(row, hbuf, sbuf, sem):
        pltpu.make_async_copy(h_hbm.at[row], hbuf.at[pl.ds(0, ROW_LEN)], sem).wait()
        pltpu.make_async_copy(s_hbm.at[row], sbuf.at[pl.ds(0, ROW_LEN)], sem).wait()

    def process_row(i2, hbuf, sbuf, acc0, acc1):
        def pass_a(_, carry):
            acc, vidx = carry
            hv = [plsc.load_gather(hbuf, [vidx + u]) for u in range(UNROLL)]
            xv = [v * v for v in hv]
            s01 = xv[0] + xv[1]
            s23 = xv[2] + xv[3]
            return acc + ((s01 + s23) + xv[4]), vidx + UNROLL

        seg_sums, _ = lax.fori_loop(0, NUM_STEPS, pass_a, (zeros16, vidx0))
        seg_incl = plsc.cumsum(seg_sums)
        seg_base = seg_incl - seg_sums
        a_vec = jnp.full((16,), jnp.sum(seg_sums))
        inv_vec = jnp.full((16,), 16.0) / a_vec

        binsbuf[pl.ds(0, 16)] = zeros16

        def pass_b(_, carry):
            p_run, ut, vidx = carry
            hv = [plsc.load_gather(hbuf, [vidx + u]) for u in range(UNROLL)]
            sv = [plsc.load_gather(sbuf, [vidx + u]) for u in range(UNROLL)]
            xv = [v * v for v in hv]
            s01 = xv[0] + xv[1]
            s23 = xv[2] + xv[3]
            pre = (None, xv[0], s01, s01 + xv[2], s01 + s23)
            inc = [hv[u] * sv[u] for u in range(UNROLL)]
            for u in range(UNROLL):
                p_u = p_run if u == 0 else p_run + pre[u]
                scaled = p_u * inv_vec
                k = scaled.astype(jnp.int32)
                m = scaled > 0.0
                plsc.addupdate_scatter(binsbuf, [k], inc[u], mask=m)
            i01 = inc[0] + inc[1]
            i23 = inc[2] + inc[3]
            ut = ut + ((i01 + i23) + inc[4])
            p_run = p_run + ((s01 + s23) + xv[4])
            return p_run, ut, vidx + UNROLL

        _, ut, _ = lax.fori_loop(0, NUM_STEPS, pass_b, (seg_base, zeros16, vidx0))

        u_tot = jnp.full((16,), jnp.sum(ut))
        sv_bins = binsbuf[pl.ds(0, 16)]
        dev = sv_bins - u_tot * (1.0 / 16.0)
        css = jnp.sum(dev * dev)
        chisq_b = jnp.full((16,), css) * inv_vec * ((16.0 / 15.0) / 32.0)
        acc0 = acc0 + jnp.where((iota16 == i2) & (i2 < 16), chisq_b, zeros16)
        acc1 = acc1 + jnp.where((iota16 == (i2 - 16)) & (i2 >= 16), chisq_b, zeros16)
        return acc0, acc1

    start_row(base_row, h0, s0, sem0)

    def pair_body(i, carry):
        acc0, acc1 = carry
        ra = base_row + 2 * i
        rb = ra + 1
        start_row(rb, h1, s1, sem1)
        wait_row(ra, h0, s0, sem0)
        acc0, acc1 = process_row(2 * i, h0, s0, acc0, acc1)

        @pl.when(i < (ROWS_PER_W // 2 - 1))
        def _():
            start_row(ra + 2, h0, s0, sem0)

        wait_row(rb, h1, s1, sem1)
        acc0, acc1 = process_row(2 * i + 1, h1, s1, acc0, acc1)
        return acc0, acc1

    acc0, acc1 = lax.fori_loop(0, ROWS_PER_W // 2, pair_body, (zeros16, zeros16))
    outbuf[pl.ds(0, 16)] = acc0
    outbuf[pl.ds(16, 16)] = acc1
    pltpu.sync_copy(outbuf, out_hbm.at[pl.ds(base_row, ROWS_PER_W)])


@jax.jit
def kernel(template, strain):
    h = template.reshape(NUM_ROWS, ROW_LEN)
    s = strain.reshape(NUM_ROWS, ROW_LEN)
    mesh = plsc.VectorSubcoreMesh(
        core_axis_name="c", subcore_axis_name="s", num_cores=NC, num_subcores=NS
    )
    out = pl.kernel(
        _sc_body,
        out_type=jax.ShapeDtypeStruct((NUM_ROWS,), jnp.float32),
        mesh=mesh,
        scratch_types=[
            pltpu.VMEM((PAD_LEN,), jnp.float32),
            pltpu.VMEM((PAD_LEN,), jnp.float32),
            pltpu.VMEM((PAD_LEN,), jnp.float32),
            pltpu.VMEM((PAD_LEN,), jnp.float32),
            pltpu.VMEM((32,), jnp.float32),
            pltpu.VMEM((ROWS_PER_W,), jnp.float32),
            pltpu.SemaphoreType.DMA,
            pltpu.SemaphoreType.DMA,
        ],
        compiler_params=pltpu.CompilerParams(
            needs_layout_passes=False, use_tc_tiling_on_sc=False
        ),
    )(h, s)
    return out.reshape(512, 2)
